# baseline (speedup 1.0000x reference)
import functools

import jax
import jax.numpy as jnp
from jax import lax
from jax.experimental import pallas as pl
from jax.experimental.pallas import tpu as pltpu
from jax.experimental.pallas import tpu_sc as plsc

M = 1000000
D = 64
BATCH = 16384
NVEC = BATCH // 16
IDXBLK = 4096
NC = 2
NS = 16
NW = NC * NS
CRL = 8
CR = 1 << CRL
NCHT = -(-M // CR)
TAIL = M - (NCHT - 1) * CR
TPW = -(-NCHT // NW)
POS_B = 14
T_SH = CRL + POS_B

_mesh = plsc.VectorSubcoreMesh(core_axis_name="c", subcore_axis_name="s")


@functools.partial(
    pl.kernel,
    out_type=jax.ShapeDtypeStruct((M, D), jnp.float32),
    mesh=_mesh,
    compiler_params=pltpu.CompilerParams(needs_layout_passes=False),
    scratch_types=[
        pltpu.VMEM((IDXBLK,), jnp.int32),
        pltpu.VMEM((16 * 1024,), jnp.int32),
        pltpu.VMEM((BATCH + 16,), jnp.int32),
        pltpu.VMEM((16,), jnp.int32),
        pltpu.VMEM((16,), jnp.int32),
        pltpu.VMEM((16, 2 * D), jnp.float32),
        pltpu.VMEM((CR, D), jnp.float32),
        pltpu.SMEM((TPW + 2,), jnp.int32),
        pltpu.SMEM((TPW + 2,), jnp.int32),
        pltpu.SMEM((TPW + 2,), jnp.int32),
        pltpu.SMEM((16,), jnp.int32),
    ],
)
def _scatter_add_kernel(index_hbm, a_hbm, b2_hbm, out_hbm,
                        idx_v, subs, srt, tmp, p2_ref, bbuf2, chunk,
                        cnts, seg0, cur, csm):
    cid = lax.axis_index("c")
    sid = lax.axis_index("s")
    wid = cid * NS + sid
    lane = lax.iota(jnp.int32, 16)

    c_vec = jnp.zeros((16,), jnp.int32)
    for blk in range(BATCH // IDXBLK):
        pltpu.sync_copy(index_hbm.at[pl.ds(blk * IDXBLK, IDXBLK)], idx_v)

        def scan_body(v, c_vec, blk=blk):
            vec = idx_v[pl.ds(v * 16, 16)]
            mask = ((vec >> CRL) & (NW - 1)) == wid
            t = vec >> (CRL + 5)
            code = (t << T_SH) | ((vec & (CR - 1)) << POS_B) \
                | (blk * IDXBLK + v * 16 + lane)
            plsc.store_scatter(subs, [lane * 1024 + c_vec], code, mask=mask)
            return c_vec + mask.astype(jnp.int32)

        c_vec = lax.fori_loop(0, IDXBLK // 16, scan_body, c_vec,
                              unroll=False)
    tmp[pl.ds(0, 16)] = c_vec

    c_ld = tmp[pl.ds(0, 16)]
    for l in range(16):
        csm[l] = c_ld[l]

    def zero_body(t, _):
        cnts[t] = 0
        return 0

    lax.fori_loop(0, TPW, zero_body, 0, unroll=False)

    def cnt_lane(l, _):
        c_l = csm[l]

        def cnt_body(i, _2):
            cv = subs[pl.ds(l * 1024 + i * 16, 16)]
            for k in range(16):
                @pl.when(i * 16 + k < c_l)
                def _(k=k):
                    t_k = cv[k] >> T_SH
                    cnts[t_k] = cnts[t_k] + 1
            return 0

        lax.fori_loop(0, (c_l + 15) >> 4, cnt_body, 0, unroll=False)
        return 0

    lax.fori_loop(0, 16, cnt_lane, 0, unroll=False)

    def pfx_body(t, acc):
        seg0[t] = acc
        cur[t] = acc
        return acc + cnts[t]

    lax.fori_loop(0, TPW, pfx_body, jnp.int32(0), unroll=False)

    def place_lane(l, _):
        c_l = csm[l]

        def place_body(i, _2):
            cv = subs[pl.ds(l * 1024 + i * 16, 16)]
            valid = (i * 16 + lane) < c_l
            ovec = jnp.zeros((16,), jnp.int32)
            for k in range(16):
                def take(k=k, cv=cv, ovec=ovec):
                    t_k = cv[k] >> T_SH
                    o = cur[t_k]
                    cur[t_k] = o + 1
                    return jnp.where(lane == k, o, ovec)

                ovec = lax.cond(i * 16 + k < c_l, take,
                                lambda ovec=ovec: ovec)
            plsc.store_scatter(srt, [ovec], cv, mask=valid)
            return 0

        lax.fori_loop(0, (c_l + 15) >> 4, place_body, 0, unroll=False)
        return 0

    lax.fori_loop(0, 16, place_lane, 0, unroll=False)

    def trip_body(t, _):
        j = wid + t * NW

        def do_chunk():
            base_c = j * CR
            lax.cond(
                j == NCHT - 1,
                lambda: pltpu.sync_copy(a_hbm.at[pl.ds(base_c, TAIL)],
                                        chunk.at[pl.ds(0, TAIL)]),
                lambda: pltpu.sync_copy(a_hbm.at[pl.ds(base_c, CR)],
                                        chunk.at[pl.ds(0, CR)]),
            )
            cnt_t = cnts[t]
            s_t = seg0[t]

            def grp_body(g, _2):
                base_g = s_t + g * 16
                cv = srt[pl.ds(base_g, 16)]
                vcnt = jnp.minimum(cnt_t - g * 16, 16)
                pos = cv & ((1 << POS_B) - 1)
                p2_ref[pl.ds(0, 16)] = jnp.where(lane < vcnt, pos >> 1, lane)
                pltpu.sync_copy(b2_hbm.at[p2_ref], bbuf2)
                for k in range(16):
                    @pl.when(k < vcnt)
                    def _(k=k):
                        c_k = cv[k]
                        lr = (c_k >> POS_B) & (CR - 1)
                        h = (c_k & 1) * D
                        for c in range(0, D, 16):
                            plsc.addupdate(
                                chunk.at[lr, pl.ds(c, 16)],
                                bbuf2[k, pl.ds(h + c, 16)])
                return 0

            lax.fori_loop(0, (cnt_t + 15) >> 4, grp_body, 0, unroll=False)
            lax.cond(
                j == NCHT - 1,
                lambda: pltpu.sync_copy(chunk.at[pl.ds(0, TAIL)],
                                        out_hbm.at[pl.ds(base_c, TAIL)]),
                lambda: pltpu.sync_copy(chunk.at[pl.ds(0, CR)],
                                        out_hbm.at[pl.ds(base_c, CR)]),
            )

        lax.cond(j < NCHT, do_chunk, lambda: None)
        return 0

    lax.fori_loop(0, TPW, trip_body, 0, unroll=False)


def kernel(index, A, B):
    return _scatter_add_kernel(index.astype(jnp.int32), A,
                               B.reshape(BATCH // 2, 2 * D))
